# baseline (speedup 1.0000x reference)
import jax
import jax.numpy as jnp
from jax.experimental import pallas as pl
from jax.experimental.pallas import tpu as pltpu

ROWS = 4096
COLS = 16384
R_BLK = 128
HALF = COLS // 2
CHUNK = 128
NBUF = 4
NUNITS = (ROWS // R_BLK) * 2


def _cumsum_kernel(x_hbm, t3_ref, out_hbm, in_buf, out_buf, in_sem, out_sem):
    t3 = t3_ref[...]

    def in_copy(u, b):
        r, h = u // 2, u % 2
        return pltpu.make_async_copy(
            x_hbm.at[pl.ds(r * R_BLK, R_BLK), pl.ds(h * HALF, HALF)],
            in_buf.at[b], in_sem.at[b])

    def out_copy(u, b):
        r, h = u // 2, u % 2
        return pltpu.make_async_copy(
            out_buf.at[b],
            out_hbm.at[pl.ds(r * R_BLK, R_BLK), pl.ds(h * HALF, HALF)],
            out_sem.at[b])

    for u0 in range(NBUF):
        in_copy(u0, u0).start()

    def body(u, carry):
        b = u % NBUF
        carry = jnp.where(u % 2 == 0, jnp.zeros_like(carry), carry)
        in_copy(u, b).wait()

        @pl.when(u >= NBUF)
        def _reuse_wait():
            out_copy(u - NBUF, b).wait()

        for c in range(HALF // CHUNK):
            xc = in_buf[b, :, c * CHUNK:(c + 1) * CHUNK]
            hi = xc.astype(jnp.bfloat16)
            lo = (xc - hi.astype(jnp.float32)).astype(jnp.bfloat16)
            hl = jnp.concatenate([hi, lo], axis=1)
            res = jnp.dot(hl, t3, preferred_element_type=jnp.float32)
            out_buf[b, :, c * CHUNK:(c + 1) * CHUNK] = res[:, :CHUNK] + carry
            carry = carry + res[:, CHUNK:]
        out_copy(u, b).start()

        @pl.when(u + NBUF < NUNITS)
        def _prefetch():
            in_copy(u + NBUF, b).start()

        return carry

    jax.lax.fori_loop(0, NUNITS, body,
                      jnp.zeros((R_BLK, CHUNK), jnp.float32))
    for k in range(NBUF):
        u = NUNITS - NBUF + k
        out_copy(u, u % NBUF).wait()


@jax.jit
def kernel(x):
    tri = jnp.triu(jnp.ones((CHUNK, CHUNK), dtype=jnp.bfloat16))
    t2 = jnp.concatenate(
        [tri, jnp.ones((CHUNK, CHUNK), dtype=jnp.bfloat16)], axis=1)
    t3 = jnp.concatenate([t2, t2], axis=0)
    return pl.pallas_call(
        _cumsum_kernel,
        in_specs=[
            pl.BlockSpec(memory_space=pltpu.MemorySpace.HBM),
            pl.BlockSpec(memory_space=pltpu.MemorySpace.VMEM),
        ],
        out_specs=pl.BlockSpec(memory_space=pltpu.MemorySpace.HBM),
        out_shape=jax.ShapeDtypeStruct((ROWS, COLS), jnp.float32),
        scratch_shapes=[
            pltpu.VMEM((NBUF, R_BLK, HALF), jnp.float32),
            pltpu.VMEM((NBUF, R_BLK, HALF), jnp.float32),
            pltpu.SemaphoreType.DMA((NBUF,)),
            pltpu.SemaphoreType.DMA((NBUF,)),
        ],
    )(x, t3)
